# baseline (speedup 1.0000x reference)
import jax
import jax.numpy as jnp
from jax import lax
from jax.experimental import pallas as pl
from jax.experimental.pallas import tpu as pltpu
from jax.experimental.pallas import tpu_sc as plsc

VOCAB = 1000000
N_EMBD = 64
B, L = 4096, 200

NW = 32
LB = 128
VOC2 = VOCAB // 2
NBUF = 2


def _gather_body(xt_hbm, tab_hbm, out_hbm, idx_v, g_v, rows_v, ot_v, gsems, osem):
    c = lax.axis_index("c")
    s = lax.axis_index("s")
    wid = s * 2 + c
    bbase = wid * LB

    pltpu.sync_copy(xt_hbm.at[:, pl.ds(bbase, LB)], idx_v)

    @pl.loop(0, L)
    def _shift(l):
        for cc in range(LB // 16):
            v = idx_v[l, pl.ds(cc * 16, 16)]
            g_v[l, pl.ds(cc * 16, 16)] = lax.shift_right_logical(v, 1)

    def fire(b, l):
        pltpu.async_copy(tab_hbm.at[g_v.at[l]], rows_v.at[b], gsems[b])

    def drain(b):
        pltpu.make_async_copy(
            tab_hbm.at[pl.ds(0, LB)], rows_v.at[b], gsems[b]
        ).wait()

    iota16 = lax.iota(jnp.int32, 16)

    def transpose_store(b, l):
        pcols = [
            lax.shift_left(
                lax.bitwise_and(idx_v[l, pl.ds(j0 * 16, 16)], 1), 6
            )
            for j0 in range(LB // 16)
        ]
        jvs = [iota16 + (j0 * 16) for j0 in range(LB // 16)]

        @pl.loop(0, 16)
        def _d(d):
            ed = lax.bitwise_and(iota16 + d, 15)
            for e0 in range(N_EMBD // 16):
                ev = ed + (e0 * 16)
                for j0 in range(LB // 16):
                    cv = pcols[j0] + ev
                    vals = plsc.load_gather(rows_v.at[b], [jvs[j0], cv])
                    plsc.store_scatter(ot_v, [ev, jvs[j0]], vals)

    for b in range(NBUF):
        fire(b, b)

    @pl.loop(0, (L - NBUF) // NBUF)
    def _t(t):
        for b in range(NBUF):
            l = t * NBUF + b
            drain(b)
            transpose_store(b, l)
            fire(b, l + NBUF)
            cp = pltpu.async_copy(
                ot_v, out_hbm.at[l, :, pl.ds(bbase, LB)], osem
            )
            cp.wait()

    for b in range(NBUF):
        l = L - NBUF + b
        drain(b)
        transpose_store(b, l)
        pltpu.async_copy(ot_v, out_hbm.at[l, :, pl.ds(bbase, LB)], osem).wait()


@jax.jit
def _embed_lookup(xt, tab_pairs):
    mesh = plsc.VectorSubcoreMesh(core_axis_name="c", subcore_axis_name="s")
    return pl.kernel(
        _gather_body,
        out_type=jax.ShapeDtypeStruct((L, N_EMBD, B), jnp.float32),
        mesh=mesh,
        scratch_types=[
            pltpu.VMEM((L, LB), jnp.int32),
            pltpu.VMEM((L, LB), jnp.int32),
            pltpu.VMEM((NBUF, LB, 128), jnp.float32),
            pltpu.VMEM((N_EMBD, LB), jnp.float32),
            [pltpu.SemaphoreType.DMA] * NBUF,
            pltpu.SemaphoreType.DMA,
        ],
        compiler_params=pltpu.CompilerParams(needs_layout_passes=False),
    )(xt, tab_pairs)


def kernel(x, token_embed):
    xt = x.astype(jnp.int32).T
    tab_pairs = (
        token_embed.T.reshape(N_EMBD, VOC2, 2).transpose(1, 2, 0).reshape(VOC2, 128)
    )
    out_t = _embed_lookup(xt, tab_pairs)
    return out_t.transpose(2, 0, 1)


# speedup vs baseline: 1.2191x; 1.2191x over previous
import jax
import jax.numpy as jnp
from jax import lax
from jax.experimental import pallas as pl
from jax.experimental.pallas import tpu as pltpu
from jax.experimental.pallas import tpu_sc as plsc

VOCAB = 1000000
N_EMBD = 64
B, L = 4096, 200

NW = 32
LB = 128
VOC2 = VOCAB // 2
NBUF = 2


def _gather_body(xt_hbm, tab_hbm, out_hbm, idx_v, g_v, rows_v, ot_v, gsems, osems):
    c = lax.axis_index("c")
    s = lax.axis_index("s")
    wid = s * 2 + c
    bbase = wid * LB

    pltpu.sync_copy(xt_hbm.at[:, pl.ds(bbase, LB)], idx_v)

    @pl.loop(0, L)
    def _shift(l):
        for cc in range(LB // 16):
            v = idx_v[l, pl.ds(cc * 16, 16)]
            g_v[l, pl.ds(cc * 16, 16)] = lax.shift_right_logical(v, 1)

    def fire(b, l):
        pltpu.async_copy(tab_hbm.at[g_v.at[l]], rows_v.at[b], gsems[b])

    def drain(b):
        pltpu.make_async_copy(
            tab_hbm.at[pl.ds(0, LB)], rows_v.at[b], gsems[b]
        ).wait()

    iota16 = lax.iota(jnp.int32, 16)

    def transpose_store(b, l):
        pcols = [
            lax.shift_left(
                lax.bitwise_and(idx_v[l, pl.ds(j0 * 16, 16)], 1), 6
            )
            for j0 in range(LB // 16)
        ]
        jvs = [iota16 + (j0 * 16) for j0 in range(LB // 16)]

        @pl.loop(0, 16)
        def _d(d):
            ed = lax.bitwise_and(iota16 + d, 15)
            for e0 in range(N_EMBD // 16):
                ev = ed + (e0 * 16)
                for j0 in range(LB // 16):
                    cv = pcols[j0] + ev
                    vals = plsc.load_gather(rows_v.at[b], [jvs[j0], cv])
                    plsc.store_scatter(ot_v.at[b], [ev, jvs[j0]], vals)

    def drain_write(b):
        pltpu.make_async_copy(
            out_hbm.at[0, :, pl.ds(bbase, LB)], ot_v.at[b], osems[b]
        ).wait()

    def stage(l, b, wait_write, fire_next):
        drain(b)
        if wait_write:
            drain_write(b)
        transpose_store(b, l)
        if fire_next:
            fire(b, l + NBUF)
        pltpu.async_copy(ot_v.at[b], out_hbm.at[l, :, pl.ds(bbase, LB)], osems[b])

    for b in range(NBUF):
        fire(b, b)
    for b in range(NBUF):
        stage(b, b, False, True)

    @pl.loop(1, (L - NBUF) // NBUF)
    def _t(t):
        for b in range(NBUF):
            stage(t * NBUF + b, b, True, True)

    for b in range(NBUF):
        stage(L - NBUF + b, b, True, False)
    for b in range(NBUF):
        drain_write(b)


@jax.jit
def _embed_lookup(xt, tab_pairs):
    mesh = plsc.VectorSubcoreMesh(core_axis_name="c", subcore_axis_name="s")
    return pl.kernel(
        _gather_body,
        out_type=jax.ShapeDtypeStruct((L, N_EMBD, B), jnp.float32),
        mesh=mesh,
        scratch_types=[
            pltpu.VMEM((L, LB), jnp.int32),
            pltpu.VMEM((L, LB), jnp.int32),
            pltpu.VMEM((NBUF, LB, 128), jnp.float32),
            pltpu.VMEM((NBUF, N_EMBD, LB), jnp.float32),
            [pltpu.SemaphoreType.DMA] * NBUF,
            [pltpu.SemaphoreType.DMA] * NBUF,
        ],
        compiler_params=pltpu.CompilerParams(needs_layout_passes=False),
    )(xt, tab_pairs)


def kernel(x, token_embed):
    xt = x.astype(jnp.int32).T
    tab_pairs = token_embed.reshape(VOC2, 128)
    out_t = _embed_lookup(xt, tab_pairs)
    return out_t.transpose(2, 0, 1)


# speedup vs baseline: 1.2223x; 1.0027x over previous
import jax
import jax.numpy as jnp
from jax import lax
from jax.experimental import pallas as pl
from jax.experimental.pallas import tpu as pltpu
from jax.experimental.pallas import tpu_sc as plsc

VOCAB = 1000000
N_EMBD = 64
B, L = 4096, 200

NW = 32
LB = 128
VOC2 = VOCAB // 2
NBUF = 2


def _gather_body(xt_hbm, tab_hbm, out_hbm, idx_v, g_v, rows_v, ot_v, gsems, osems):
    c = lax.axis_index("c")
    s = lax.axis_index("s")
    wid = s * 2 + c
    bbase = wid * LB

    pltpu.sync_copy(xt_hbm.at[:, pl.ds(bbase, LB)], idx_v)

    @pl.loop(0, L)
    def _shift(l):
        for cc in range(LB // 16):
            v = idx_v[l, pl.ds(cc * 16, 16)]
            g_v[l, pl.ds(cc * 16, 16)] = lax.shift_right_logical(v, 1)

    def fire(b, l):
        for h in range(2):
            pltpu.async_copy(
                tab_hbm.at[g_v.at[l, pl.ds(h * 64, 64)]],
                rows_v.at[b, pl.ds(h * 64, 64)],
                gsems[b],
            )

    def drain(b):
        pltpu.make_async_copy(
            tab_hbm.at[pl.ds(0, LB)], rows_v.at[b], gsems[b]
        ).wait()

    iota16 = lax.iota(jnp.int32, 16)

    def transpose_store(b, l):
        pcols = [
            lax.shift_left(
                lax.bitwise_and(idx_v[l, pl.ds(j0 * 16, 16)], 1), 6
            )
            for j0 in range(LB // 16)
        ]
        jvs = [iota16 + (j0 * 16) for j0 in range(LB // 16)]

        @pl.loop(0, 16)
        def _d(d):
            ed = lax.bitwise_and(iota16 + d, 15)
            for e0 in range(N_EMBD // 16):
                ev = ed + (e0 * 16)
                for j0 in range(LB // 16):
                    cv = pcols[j0] + ev
                    vals = plsc.load_gather(rows_v.at[b], [jvs[j0], cv])
                    plsc.store_scatter(ot_v.at[b], [ev, jvs[j0]], vals)

    def drain_write(b):
        pltpu.make_async_copy(
            out_hbm.at[0, :, pl.ds(bbase, LB)], ot_v.at[b], osems[b]
        ).wait()

    def stage(l, b, wait_write, fire_next):
        drain(b)
        if wait_write:
            drain_write(b)
        transpose_store(b, l)
        if fire_next:
            fire(b, l + NBUF)
        pltpu.async_copy(ot_v.at[b], out_hbm.at[l, :, pl.ds(bbase, LB)], osems[b])

    for b in range(NBUF):
        fire(b, b)
    for b in range(NBUF):
        stage(b, b, False, True)

    @pl.loop(1, (L - NBUF) // NBUF)
    def _t(t):
        for b in range(NBUF):
            stage(t * NBUF + b, b, True, True)

    for b in range(NBUF):
        stage(L - NBUF + b, b, True, False)
    for b in range(NBUF):
        drain_write(b)


@jax.jit
def _embed_lookup(xt, tab_pairs):
    mesh = plsc.VectorSubcoreMesh(core_axis_name="c", subcore_axis_name="s")
    return pl.kernel(
        _gather_body,
        out_type=jax.ShapeDtypeStruct((L, N_EMBD, B), jnp.float32),
        mesh=mesh,
        scratch_types=[
            pltpu.VMEM((L, LB), jnp.int32),
            pltpu.VMEM((L, LB), jnp.int32),
            pltpu.VMEM((NBUF, LB, 128), jnp.float32),
            pltpu.VMEM((NBUF, N_EMBD, LB), jnp.float32),
            [pltpu.SemaphoreType.DMA] * NBUF,
            [pltpu.SemaphoreType.DMA] * NBUF,
        ],
        compiler_params=pltpu.CompilerParams(needs_layout_passes=False),
    )(xt, tab_pairs)


def kernel(x, token_embed):
    xt = x.astype(jnp.int32).T
    tab_pairs = token_embed.reshape(VOC2, 128)
    out_t = _embed_lookup(xt, tab_pairs)
    return out_t.transpose(2, 0, 1)


# speedup vs baseline: 1.3328x; 1.0904x over previous
import jax
import jax.numpy as jnp
from jax import lax
from jax.experimental import pallas as pl
from jax.experimental.pallas import tpu as pltpu
from jax.experimental.pallas import tpu_sc as plsc

VOCAB = 1000000
N_EMBD = 64
B, L = 4096, 200

NW = 32
LB = 128
VOC2 = VOCAB // 2
NBUF = 2


def _gather_body(xt_hbm, tab_hbm, out_hbm, idx_v, g_v, rows_v, ot_v, gsems, osems):
    c = lax.axis_index("c")
    s = lax.axis_index("s")
    wid = s * 2 + c
    bbase = wid * LB

    pltpu.sync_copy(xt_hbm.at[:, pl.ds(bbase, LB)], idx_v)

    @pl.loop(0, L)
    def _shift(l):
        for cc in range(LB // 16):
            v = idx_v[l, pl.ds(cc * 16, 16)]
            g_v[l, pl.ds(cc * 16, 16)] = lax.shift_right_logical(v, 1)

    def fire(b, l):
        for h in range(2):
            pltpu.async_copy(
                tab_hbm.at[g_v.at[l, pl.ds(h * 64, 64)]],
                rows_v.at[b, pl.ds(h * 64, 64)],
                gsems[b],
            )

    def drain(b):
        pltpu.make_async_copy(
            tab_hbm.at[pl.ds(0, LB)], rows_v.at[b], gsems[b]
        ).wait()

    iota16 = lax.iota(jnp.int32, 16)

    def transpose_store(b, l):
        pcols = [
            lax.shift_left(
                lax.bitwise_and(idx_v[l, pl.ds(j0 * 16, 16)], 1), 6
            )
            for j0 in range(LB // 16)
        ]
        jvs = [iota16 + (j0 * 16) for j0 in range(LB // 16)]

        @pl.loop(0, 16)
        def _d(d):
            ed = lax.bitwise_and(iota16 + d, 15)
            for e0 in range(N_EMBD // 16):
                ev = ed + (e0 * 16)
                for j0 in range(LB // 16):
                    cv = pcols[j0] + ev
                    vals = plsc.load_gather(rows_v.at[b], [jvs[j0], cv])
                    plsc.store_scatter(ot_v.at[b], [ev, jvs[j0]], vals)

    def drain_write(b):
        pltpu.make_async_copy(
            out_hbm.at[0, :, pl.ds(bbase, LB)], ot_v.at[b], osems[b]
        ).wait()

    def stage(l, b, wait_write, fire_next):
        drain(b)
        if wait_write:
            drain_write(b)
        transpose_store(b, l)
        if fire_next:
            fire(b, l + NBUF)
        pltpu.async_copy(ot_v.at[b], out_hbm.at[l, :, pl.ds(bbase, LB)], osems[b])

    for b in range(NBUF):
        fire(b, b)
    for b in range(NBUF):
        stage(b, b, False, True)

    @pl.loop(1, (L - NBUF) // NBUF)
    def _t(t):
        for b in range(NBUF):
            stage(t * NBUF + b, b, True, True)

    for b in range(NBUF):
        stage(L - NBUF + b, b, True, False)
    for b in range(NBUF):
        drain_write(b)


NCH = 7812
CPW = NCH // NW


def _prep_body(tabt_hbm, tail_hbm, pt_hbm, blk_v, otb_v, isems, osems):
    c_ax = lax.axis_index("c")
    s_ax = lax.axis_index("s")
    wid = s_ax * 2 + c_ax

    iota16 = lax.iota(jnp.int32, 16)
    cvb = [lax.mul(iota16, 2) + (q0 * 32) for q0 in range(4)]

    def chunk_of(k):
        return wid + k * NW

    def fire_in(b, k, width=128):
        pltpu.async_copy(
            tabt_hbm.at[:, pl.ds(chunk_of(k) * 128, width)],
            blk_v.at[b, :, pl.ds(0, width)],
            isems[b],
        )

    def drain_in(b, width=128):
        pltpu.make_async_copy(
            tabt_hbm.at[:, pl.ds(0, width)], blk_v.at[b, :, pl.ds(0, width)],
            isems[b],
        ).wait()

    def drain_out(b, rows=64):
        pltpu.make_async_copy(
            pt_hbm.at[pl.ds(0, rows)], otb_v.at[b, pl.ds(0, rows)], osems[b]
        ).wait()

    def transpose(b, nq0=4):
        @pl.loop(0, 16)
        def _d(d):
            zd = lax.bitwise_and(iota16 + d, 15)
            for p in range(2):
                for z0 in range(4):
                    ev = zd + (z0 * 16)
                    zov = ev + (p * 64)
                    for q0 in range(nq0):
                        cv = cvb[q0] + p
                        vals = plsc.load_gather(blk_v.at[b], [ev, cv])
                        plsc.store_scatter(
                            otb_v.at[b], [iota16 + (q0 * 16), zov], vals
                        )

    def stage(k, b, wait_write, fire_next):
        drain_in(b)
        if wait_write:
            drain_out(b)
        transpose(b)
        if fire_next:
            fire_in(b, k + 2)
        pltpu.async_copy(
            otb_v.at[b], pt_hbm.at[pl.ds(chunk_of(k) * 64, 64)], osems[b]
        )

    for b in range(2):
        fire_in(b, b)
    for b in range(2):
        stage(b, b, False, True)

    @pl.loop(1, CPW // 2 - 1)
    def _t(t):
        for b in range(2):
            stage(t * 2 + b, b, True, True)

    for b in range(2):
        stage(CPW - 2 + b, b, True, False)
    for b in range(2):
        drain_out(b)

    @pl.when(wid < 4)
    def _extra():
        c = NCH - 4 + wid
        pltpu.sync_copy(tabt_hbm.at[:, pl.ds(c * 128, 128)], blk_v.at[0])
        transpose(0)
        pltpu.sync_copy(otb_v.at[0], pt_hbm.at[pl.ds(c * 64, 64)])

    @pl.when(wid == 4)
    def _tail():
        pltpu.sync_copy(tail_hbm, blk_v.at[0])
        transpose(0, nq0=2)
        pltpu.sync_copy(
            otb_v.at[0, pl.ds(0, 32)], pt_hbm.at[pl.ds(NCH * 64, 32)]
        )


@jax.jit
def _pair_table(tabt, tail):
    mesh = plsc.VectorSubcoreMesh(core_axis_name="c", subcore_axis_name="s")
    return pl.kernel(
        _prep_body,
        out_type=jax.ShapeDtypeStruct((VOC2, 128), jnp.float32),
        mesh=mesh,
        scratch_types=[
            pltpu.VMEM((2, N_EMBD, 128), jnp.float32),
            pltpu.VMEM((2, N_EMBD, 128), jnp.float32),
            [pltpu.SemaphoreType.DMA] * 2,
            [pltpu.SemaphoreType.DMA] * 2,
        ],
        compiler_params=pltpu.CompilerParams(needs_layout_passes=False),
    )(tabt, tail)


@jax.jit
def _embed_lookup(xt, tab_pairs):
    mesh = plsc.VectorSubcoreMesh(core_axis_name="c", subcore_axis_name="s")
    return pl.kernel(
        _gather_body,
        out_type=jax.ShapeDtypeStruct((L, N_EMBD, B), jnp.float32),
        mesh=mesh,
        scratch_types=[
            pltpu.VMEM((L, LB), jnp.int32),
            pltpu.VMEM((L, LB), jnp.int32),
            pltpu.VMEM((NBUF, LB, 128), jnp.float32),
            pltpu.VMEM((NBUF, N_EMBD, LB), jnp.float32),
            [pltpu.SemaphoreType.DMA] * NBUF,
            [pltpu.SemaphoreType.DMA] * NBUF,
        ],
        compiler_params=pltpu.CompilerParams(needs_layout_passes=False),
    )(xt, tab_pairs)


def kernel(x, token_embed):
    xt = x.astype(jnp.int32).T
    tail = jnp.concatenate(
        [token_embed[NCH * 128:].T, jnp.zeros((N_EMBD, 64), jnp.float32)], axis=1
    )
    tab_pairs = _pair_table(token_embed.T, tail)
    out_t = _embed_lookup(xt, tab_pairs)
    return out_t.transpose(2, 0, 1)
